# Initial kernel scaffold; baseline (speedup 1.0000x reference)
import jax
import jax.numpy as jnp
from jax.experimental import pallas as pl


def kernel(x, adj_t, W0, b0, g0, be0, W1, b1, g1, be1, W2, b2, g2, be2, lw1, lb1, lw2, lb2):
    raise NotImplementedError("write your pallas kernel here")


# speedup vs baseline: 13.5321x; 13.5321x over previous
import functools

import jax
import jax.numpy as jnp
from jax import lax
from jax.experimental import pallas as pl
from jax.experimental.pallas import tpu as pltpu
from jax.experimental.pallas import tpu_sc as plsc

N = 10000
E = 320000
H = 128
C_OUT = 64
EPS = 1e-5

NB = 128
NCH = E // NB
NC = 2
NS = 16
NW = NC * NS
NP = 10240
ROWS_PER_TILE = NP // NS

_mesh = plsc.VectorSubcoreMesh(core_axis_name="c", subcore_axis_name="s")
_f32 = jnp.float32


def _chunk_count(wid):
    return jnp.where(wid < NCH % NW, NCH // NW + 1, NCH // NW)


@functools.partial(
    pl.kernel,
    out_type=jax.ShapeDtypeStruct((NC * NP, 8), _f32),
    mesh=_mesh,
    scratch_types=[
        pltpu.VMEM((NB,), jnp.int32),
        pltpu.VMEM((NB, 8), _f32),
        pltpu.VMEM_SHARED((NP, 8), _f32),
    ],
)
def _deg_kernel(dst_hbm, zeros_hbm, ones_hbm, out_hbm, di_v, ones_v, acc):
    cid = lax.axis_index("c")
    sid = lax.axis_index("s")
    wid = sid * NC + cid
    r0 = sid * ROWS_PER_TILE
    pltpu.sync_copy(ones_hbm, ones_v)
    pltpu.sync_copy(zeros_hbm.at[pl.ds(r0, ROWS_PER_TILE)],
                    acc.at[pl.ds(r0, ROWS_PER_TILE)])
    plsc.subcore_barrier()

    def body(i, carry):
        off = (wid + i * NW) * NB
        pltpu.sync_copy(dst_hbm.at[pl.ds(off, NB)], di_v)
        pltpu.sync_copy(ones_v, acc.at[di_v], add=True)
        return carry

    lax.fori_loop(0, _chunk_count(wid), body, 0)
    plsc.subcore_barrier()
    pltpu.sync_copy(acc.at[pl.ds(r0, ROWS_PER_TILE)],
                    out_hbm.at[pl.ds(cid * NP + r0, ROWS_PER_TILE)])


@functools.partial(
    pl.kernel,
    out_type=jax.ShapeDtypeStruct((NC * NP, H), _f32),
    mesh=_mesh,
    scratch_types=[
        pltpu.VMEM((NB,), jnp.int32),
        pltpu.VMEM((NB,), jnp.int32),
        pltpu.VMEM((NB, H), _f32),
        pltpu.VMEM_SHARED((NP, H), _f32),
        pltpu.SemaphoreType.DMA,
    ],
)
def _agg_kernel(hp_hbm, src_hbm, dst_hbm, zeros_hbm, out_hbm,
                si_v, di_v, rows_v, acc, sem):
    cid = lax.axis_index("c")
    sid = lax.axis_index("s")
    wid = sid * NC + cid
    r0 = sid * ROWS_PER_TILE
    pltpu.sync_copy(zeros_hbm.at[pl.ds(r0, ROWS_PER_TILE)],
                    acc.at[pl.ds(r0, ROWS_PER_TILE)])
    plsc.subcore_barrier()

    def body(i, carry):
        off = (wid + i * NW) * NB
        pltpu.sync_copy(src_hbm.at[pl.ds(off, NB)], si_v)
        pltpu.sync_copy(dst_hbm.at[pl.ds(off, NB)], di_v)
        pltpu.async_copy(hp_hbm.at[si_v], rows_v, sem).wait()
        pltpu.sync_copy(rows_v, acc.at[di_v], add=True)
        return carry

    lax.fori_loop(0, _chunk_count(wid), body, 0)
    plsc.subcore_barrier()
    pltpu.sync_copy(acc.at[pl.ds(r0, ROWS_PER_TILE)],
                    out_hbm.at[pl.ds(cid * NP + r0, ROWS_PER_TILE)])


def _prep_body(degp_ref, x_ref, dis_ref, hp_ref):
    deg = degp_ref[:N, 0:1] + degp_ref[NP:NP + N, 0:1] + 1.0
    dis = lax.rsqrt(deg)
    dis_ref[...] = dis
    hp_ref[...] = x_ref[...] * dis


_prep_tc = pl.pallas_call(
    _prep_body,
    out_shape=[
        jax.ShapeDtypeStruct((N, 1), _f32),
        jax.ShapeDtypeStruct((N, H), _f32),
    ],
)


def _layer_math(S_ref, hp_ref, dis_ref, W_ref, b_ref, g_ref, be_ref):
    dis = dis_ref[...]
    agg = dis * (S_ref[:N] + S_ref[NP:NP + N] + hp_ref[...])
    z = jnp.dot(agg, W_ref[...], preferred_element_type=_f32) + b_ref[...]
    mu = jnp.mean(z, axis=0, keepdims=True)
    d = z - mu
    var = jnp.mean(d * d, axis=0, keepdims=True)
    y = d * lax.rsqrt(var + EPS) * g_ref[...] + be_ref[...]
    return jnp.maximum(y, 0.0), dis


def _layer_body(S_ref, hp_ref, dis_ref, W_ref, b_ref, g_ref, be_ref,
                y_ref, hpn_ref):
    y, dis = _layer_math(S_ref, hp_ref, dis_ref, W_ref, b_ref, g_ref, be_ref)
    y_ref[...] = y
    hpn_ref[...] = y * dis


_layer_tc = pl.pallas_call(
    _layer_body,
    out_shape=[
        jax.ShapeDtypeStruct((N, H), _f32),
        jax.ShapeDtypeStruct((N, H), _f32),
    ],
)


def _final_body(S_ref, hp_ref, dis_ref, W_ref, b_ref, g_ref, be_ref,
                h1_ref, h2_ref, lw1_ref, lb1_ref, lw2_ref, lb2_ref, out_ref):
    h3, _ = _layer_math(S_ref, hp_ref, dis_ref, W_ref, b_ref, g_ref, be_ref)
    m = jnp.maximum(jnp.maximum(h1_ref[...], h2_ref[...]), h3)
    t = jnp.maximum(
        jnp.dot(m, lw1_ref[...], preferred_element_type=_f32) + lb1_ref[...],
        0.0)
    o = jnp.dot(t, lw2_ref[...], preferred_element_type=_f32) + lb2_ref[...]
    mx = jnp.max(o, axis=-1, keepdims=True)
    lse = jnp.log(jnp.sum(jnp.exp(o - mx), axis=-1, keepdims=True)) + mx
    out_ref[...] = o - lse


_final_tc = pl.pallas_call(
    _final_body,
    out_shape=jax.ShapeDtypeStruct((N, C_OUT), _f32),
)


def kernel(x, adj_t, W0, b0, g0, be0, W1, b1, g1, be1, W2, b2, g2, be2,
           lw1, lb1, lw2, lb2):
    src = adj_t[0]
    dst = adj_t[1]
    zeros8 = jnp.zeros((NP, 8), _f32)
    ones8 = jnp.ones((NB, 8), _f32)
    zerosH = jnp.zeros((NP, H), _f32)

    degp = _deg_kernel(dst, zeros8, ones8)
    dis, hp0 = _prep_tc(degp, x)

    S = _agg_kernel(hp0, src, dst, zerosH)
    h1, hp1 = _layer_tc(S, hp0, dis, W0, b0.reshape(1, H), g0.reshape(1, H),
                        be0.reshape(1, H))
    S = _agg_kernel(hp1, src, dst, zerosH)
    h2, hp2 = _layer_tc(S, hp1, dis, W1, b1.reshape(1, H), g1.reshape(1, H),
                        be1.reshape(1, H))
    S = _agg_kernel(hp2, src, dst, zerosH)
    out = _final_tc(S, hp2, dis, W2, b2.reshape(1, H), g2.reshape(1, H),
                    be2.reshape(1, H), h1, h2, lw1, lb1.reshape(1, H), lw2,
                    lb2.reshape(1, C_OUT))
    return out
